# Initial kernel scaffold; baseline (speedup 1.0000x reference)
import jax
import jax.numpy as jnp
from jax.experimental import pallas as pl


def kernel(d, c):
    raise NotImplementedError("write your pallas kernel here")


# speedup vs baseline: 4.6494x; 4.6494x over previous
import jax
import jax.numpy as jnp
from jax import lax
from jax.experimental import pallas as pl
from jax.experimental.pallas import tpu as pltpu

RB = 256
CAPC = 128
NUMC_PAD = 128
WIDTH = 128.0
BIG = 1e30
VTH = 1e29


def _metrics_body(nrounds_ref, cls_start_ref, cls_end_ref, fc_ref,
                  dp_all_ref, dp_blk_ref, sqcol_ref,
                  out_ref,
                  a_ref, vcur_ref, voth_ref, sacc_ref,
                  minr_ref, rpn_ref, apn_ref, rm1_ref):
    i = pl.program_id(0)
    n = dp_all_ref.shape[0]
    gstart = i * RB

    g = lax.dot_general(dp_blk_ref[...], dp_all_ref[...],
                        (((1,), (1,)), ((), ())),
                        preferred_element_type=jnp.float32)
    a = sqcol_ref[...] - 2.0 * g
    col = lax.broadcasted_iota(jnp.int32, (RB, n), 1)
    row_g = gstart + lax.broadcasted_iota(jnp.int32, (RB, n), 0)
    a_ref[...] = jnp.where(col == row_g, BIG, a)

    minr_ref[...] = jnp.full((RB, 1), 1e9, jnp.float32)
    rpn_ref[...] = jnp.zeros((RB, 1), jnp.float32)
    apn_ref[...] = jnp.zeros((RB, 1), jnp.float32)

    rows_g = gstart + lax.broadcasted_iota(jnp.int32, (RB, 1), 0)
    colc2 = lax.broadcasted_iota(jnp.int32, (1, 2 * CAPC), 1)
    lanec = lax.broadcasted_iota(jnp.int32, (1, CAPC), 1)

    def extract(q, v_ref):
        v_ref[...] = jnp.full((RB, CAPC), BIG, jnp.float32)

        def cond(cls):
            return jnp.logical_and(cls < NUMC_PAD,
                                   cls_start_ref[cls] < gstart + RB)

        def body(cls):
            s = cls_start_ref[cls]
            e = cls_end_ref[cls]
            m = e - s
            base = s + q * CAPC
            cs = jnp.minimum((base // CAPC) * CAPC, n - 2 * CAPC)
            cs = pl.multiple_of(cs, CAPC)
            win = a_ref[:, pl.ds(cs, 2 * CAPC)]
            memidx = (cs - s) + colc2
            validm = jnp.logical_and(memidx >= q * CAPC,
                                     memidx < jnp.minimum(m, (q + 1) * CAPC))
            win = jnp.where(validm, win, BIG)
            offset = base - cs
            folded = jnp.where(lanec >= offset,
                               win[:, :CAPC], win[:, CAPC:])
            inrow = jnp.logical_and(rows_g >= s, rows_g < e)
            v_ref[...] = jnp.where(inrow, folded, v_ref[...])
            rm1_ref[...] = jnp.where(inrow, (m - 1).astype(jnp.float32),
                                     rm1_ref[...])
            return cls + 1

        lax.while_loop(cond, body, fc_ref[i])

    nrounds = nrounds_ref[0]

    def round_body(q, _):
        extract(q, vcur_ref)

        sacc_ref[...] = jnp.zeros((RB, CAPC), jnp.float32)

        def q2_body(q2, _):
            extract(q2, voth_ref)

            def j_body(j, _):
                onehot = lanec == j
                vj = jnp.sum(jnp.where(onehot, vcur_ref[...], 0.0),
                             axis=1, keepdims=True)
                cnt = jnp.sum((voth_ref[...] < vj).astype(jnp.float32),
                              axis=1, keepdims=True)
                sacc_ref[...] += jnp.where(onehot, cnt, 0.0)
                return 0

            return lax.fori_loop(0, CAPC, j_body, 0)

        lax.fori_loop(0, nrounds, q2_body, 0)

        rcap = jnp.minimum(rm1_ref[...], WIDTH)

        def j_body2(j, _):
            onehot = lanec == j
            vj = jnp.sum(jnp.where(onehot, vcur_ref[...], 0.0),
                         axis=1, keepdims=True)
            cnt = jnp.sum((a_ref[...] < vj).astype(jnp.float32),
                          axis=1, keepdims=True)
            rank = cnt + 1.0
            valid = vj < VTH
            minr_ref[...] = jnp.minimum(minr_ref[...],
                                        jnp.where(valid, rank, 1e9))
            rpn_ref[...] += jnp.where(
                jnp.logical_and(valid, rank <= rcap + 0.5), 1.0, 0.0)
            sj = jnp.sum(jnp.where(onehot, sacc_ref[...], 0.0),
                         axis=1, keepdims=True)
            apn_ref[...] += jnp.where(
                jnp.logical_and(valid, rank <= WIDTH + 0.5),
                (sj + 1.0) / rank, 0.0)
            return 0

        lax.fori_loop(0, CAPC, j_body2, 0)
        return 0

    lax.fori_loop(0, nrounds, round_body, 0)

    minr = minr_ref[...]
    r1 = jnp.sum((minr <= 1.5).astype(jnp.float32))
    r5 = jnp.sum((minr <= 5.5).astype(jnp.float32))
    r10 = jnp.sum((minr <= 10.5).astype(jnp.float32))
    rp = jnp.sum(rpn_ref[...] / jnp.maximum(rm1_ref[...], 1.0))
    ap = jnp.sum(apn_ref[...]) / WIDTH

    lane = lax.broadcasted_iota(jnp.int32, (1, 128), 1)
    vec = (jnp.where(lane == 0, r1, 0.0) + jnp.where(lane == 1, r5, 0.0)
           + jnp.where(lane == 2, r10, 0.0) + jnp.where(lane == 3, rp, 0.0)
           + jnp.where(lane == 4, ap, 0.0))

    @pl.when(i == 0)
    def _():
        out_ref[...] = jnp.zeros((1, 128), jnp.float32)

    out_ref[...] += vec


@jax.jit
def kernel(d, c):
    n, dim = d.shape
    order = jnp.argsort(c)
    cp = c[order]
    dp = d[order]
    sq = jnp.sum(dp * dp, axis=1)[None, :]
    carange = jnp.arange(NUMC_PAD, dtype=cp.dtype)
    cls_start = jnp.searchsorted(cp, carange, side='left').astype(jnp.int32)
    cls_end = jnp.searchsorted(cp, carange, side='right').astype(jnp.int32)
    nb = n // RB
    fc = cp[::RB].astype(jnp.int32)
    maxm = jnp.max(cls_end - cls_start)
    nrounds = jnp.maximum((maxm + CAPC - 1) // CAPC, 1).astype(jnp.int32)[None]

    out = pl.pallas_call(
        _metrics_body,
        grid=(nb,),
        in_specs=[
            pl.BlockSpec(memory_space=pltpu.SMEM),
            pl.BlockSpec(memory_space=pltpu.SMEM),
            pl.BlockSpec(memory_space=pltpu.SMEM),
            pl.BlockSpec(memory_space=pltpu.SMEM),
            pl.BlockSpec((n, dim), lambda i: (0, 0)),
            pl.BlockSpec((RB, dim), lambda i: (i, 0)),
            pl.BlockSpec((1, n), lambda i: (0, 0)),
        ],
        out_specs=pl.BlockSpec((1, 128), lambda i: (0, 0)),
        out_shape=jax.ShapeDtypeStruct((1, 128), jnp.float32),
        scratch_shapes=[
            pltpu.VMEM((RB, n), jnp.float32),
            pltpu.VMEM((RB, CAPC), jnp.float32),
            pltpu.VMEM((RB, CAPC), jnp.float32),
            pltpu.VMEM((RB, CAPC), jnp.float32),
            pltpu.VMEM((RB, 1), jnp.float32),
            pltpu.VMEM((RB, 1), jnp.float32),
            pltpu.VMEM((RB, 1), jnp.float32),
            pltpu.VMEM((RB, 1), jnp.float32),
        ],
    )(nrounds, cls_start, cls_end, fc, dp, dp, sq)

    sums = out[0]
    return jnp.stack([sums[0], sums[1], sums[2], sums[3], sums[4]]) / n


# speedup vs baseline: 6.2653x; 1.3475x over previous
import jax
import jax.numpy as jnp
from jax import lax
from jax.experimental import pallas as pl
from jax.experimental.pallas import tpu as pltpu

RB = 256
CAPC = 128
NUMC_PAD = 128
WIDTH = 128.0
BIG = 1e30
VTH = 1e29


def _metrics_body(nrounds_ref, cls_start_ref, cls_end_ref, fc_ref, maxmb_ref,
                  dp_all_ref, dp_blk_ref, sqcol_ref,
                  out_ref,
                  a_ref, vcur_ref, voth_ref, sacc_ref,
                  minr_ref, rpn_ref, apn_ref, rm1_ref):
    i = pl.program_id(0)
    n = dp_all_ref.shape[0]
    gstart = i * RB

    g = lax.dot_general(dp_blk_ref[...], dp_all_ref[...],
                        (((1,), (1,)), ((), ())),
                        preferred_element_type=jnp.float32)
    a = sqcol_ref[...] - 2.0 * g
    col = lax.broadcasted_iota(jnp.int32, (RB, n), 1)
    row_g = gstart + lax.broadcasted_iota(jnp.int32, (RB, n), 0)
    a_ref[...] = jnp.where(col == row_g, BIG, a)

    minr_ref[...] = jnp.full((RB, 1), 1e9, jnp.float32)
    rpn_ref[...] = jnp.zeros((RB, 1), jnp.float32)
    apn_ref[...] = jnp.zeros((RB, 1), jnp.float32)

    rows_g = gstart + lax.broadcasted_iota(jnp.int32, (RB, 1), 0)
    colc2 = lax.broadcasted_iota(jnp.int32, (1, 2 * CAPC), 1)
    lanec = lax.broadcasted_iota(jnp.int32, (1, CAPC), 1)

    def extract(q, v_ref):
        v_ref[...] = jnp.full((RB, CAPC), BIG, jnp.float32)

        def cond(cls):
            return jnp.logical_and(cls < NUMC_PAD,
                                   cls_start_ref[cls] < gstart + RB)

        def body(cls):
            s = cls_start_ref[cls]
            e = cls_end_ref[cls]
            m = e - s
            base = s + q * CAPC
            cs = jnp.minimum((base // CAPC) * CAPC, n - 2 * CAPC)
            cs = pl.multiple_of(cs, CAPC)
            win = a_ref[:, pl.ds(cs, 2 * CAPC)]
            memidx = (cs - s) + colc2
            validm = jnp.logical_and(memidx >= q * CAPC,
                                     memidx < jnp.minimum(m, (q + 1) * CAPC))
            win = jnp.where(validm, win, BIG)
            offset = base - cs
            rot = pltpu.roll(win, 2 * CAPC - offset, axis=1)
            inrow = jnp.logical_and(rows_g >= s, rows_g < e)
            v_ref[...] = jnp.where(inrow, rot[:, :CAPC], v_ref[...])
            rm1_ref[...] = jnp.where(inrow, (m - 1).astype(jnp.float32),
                                     rm1_ref[...])
            return cls + 1

        lax.while_loop(cond, body, fc_ref[i])

    nrounds = nrounds_ref[0]
    maxmb = maxmb_ref[i]

    def round_body(q, _):
        numj = jnp.clip(maxmb - q * CAPC, 0, CAPC)
        extract(q, vcur_ref)

        sacc_ref[...] = jnp.zeros((RB, CAPC), jnp.float32)

        def q2_body(q2, _):
            extract(q2, voth_ref)

            def j_body(j, _):
                onehot = lanec == j
                vj = jnp.sum(jnp.where(onehot, vcur_ref[...], 0.0),
                             axis=1, keepdims=True)
                cnt = jnp.sum((voth_ref[...] < vj).astype(jnp.float32),
                              axis=1, keepdims=True)
                sacc_ref[...] += jnp.where(onehot, cnt, 0.0)
                return 0

            return lax.fori_loop(0, numj, j_body, 0)

        lax.fori_loop(0, nrounds, q2_body, 0)

        rcap = jnp.minimum(rm1_ref[...], WIDTH)

        def j_body2(j, _):
            onehot = lanec == j
            vj = jnp.sum(jnp.where(onehot, vcur_ref[...], 0.0),
                         axis=1, keepdims=True)
            cnt = jnp.sum((a_ref[...] < vj).astype(jnp.float32),
                          axis=1, keepdims=True)
            rank = cnt + 1.0
            valid = vj < VTH
            minr_ref[...] = jnp.minimum(minr_ref[...],
                                        jnp.where(valid, rank, 1e9))
            rpn_ref[...] += jnp.where(
                jnp.logical_and(valid, rank <= rcap + 0.5), 1.0, 0.0)
            sj = jnp.sum(jnp.where(onehot, sacc_ref[...], 0.0),
                         axis=1, keepdims=True)
            apn_ref[...] += jnp.where(
                jnp.logical_and(valid, rank <= WIDTH + 0.5),
                (sj + 1.0) / rank, 0.0)
            return 0

        lax.fori_loop(0, numj, j_body2, 0)
        return 0

    lax.fori_loop(0, nrounds, round_body, 0)

    minr = minr_ref[...]
    r1 = jnp.sum((minr <= 1.5).astype(jnp.float32))
    r5 = jnp.sum((minr <= 5.5).astype(jnp.float32))
    r10 = jnp.sum((minr <= 10.5).astype(jnp.float32))
    rp = jnp.sum(rpn_ref[...] / jnp.maximum(rm1_ref[...], 1.0))
    ap = jnp.sum(apn_ref[...]) / WIDTH

    lane = lax.broadcasted_iota(jnp.int32, (1, 128), 1)
    vec = (jnp.where(lane == 0, r1, 0.0) + jnp.where(lane == 1, r5, 0.0)
           + jnp.where(lane == 2, r10, 0.0) + jnp.where(lane == 3, rp, 0.0)
           + jnp.where(lane == 4, ap, 0.0))

    @pl.when(i == 0)
    def _():
        out_ref[...] = jnp.zeros((1, 128), jnp.float32)

    out_ref[...] += vec


@jax.jit
def kernel(d, c):
    n, dim = d.shape
    order = jnp.argsort(c)
    cp = c[order]
    dp = d[order]
    sq = jnp.sum(dp * dp, axis=1)[None, :]
    carange = jnp.arange(NUMC_PAD, dtype=cp.dtype)
    cls_start = jnp.searchsorted(cp, carange, side='left').astype(jnp.int32)
    cls_end = jnp.searchsorted(cp, carange, side='right').astype(jnp.int32)
    nb = n // RB
    fc = cp[::RB].astype(jnp.int32)
    m_row = (cls_end - cls_start)[cp]
    maxmb = jnp.max(m_row.reshape(nb, RB), axis=1).astype(jnp.int32)
    maxm = jnp.max(cls_end - cls_start)
    nrounds = jnp.maximum((maxm + CAPC - 1) // CAPC, 1).astype(jnp.int32)[None]

    out = pl.pallas_call(
        _metrics_body,
        grid=(nb,),
        in_specs=[
            pl.BlockSpec(memory_space=pltpu.SMEM),
            pl.BlockSpec(memory_space=pltpu.SMEM),
            pl.BlockSpec(memory_space=pltpu.SMEM),
            pl.BlockSpec(memory_space=pltpu.SMEM),
            pl.BlockSpec(memory_space=pltpu.SMEM),
            pl.BlockSpec((n, dim), lambda i: (0, 0)),
            pl.BlockSpec((RB, dim), lambda i: (i, 0)),
            pl.BlockSpec((1, n), lambda i: (0, 0)),
        ],
        out_specs=pl.BlockSpec((1, 128), lambda i: (0, 0)),
        out_shape=jax.ShapeDtypeStruct((1, 128), jnp.float32),
        scratch_shapes=[
            pltpu.VMEM((RB, n), jnp.float32),
            pltpu.VMEM((RB, CAPC), jnp.float32),
            pltpu.VMEM((RB, CAPC), jnp.float32),
            pltpu.VMEM((RB, CAPC), jnp.float32),
            pltpu.VMEM((RB, 1), jnp.float32),
            pltpu.VMEM((RB, 1), jnp.float32),
            pltpu.VMEM((RB, 1), jnp.float32),
            pltpu.VMEM((RB, 1), jnp.float32),
        ],
    )(nrounds, cls_start, cls_end, fc, maxmb, dp, dp, sq)

    sums = out[0]
    return jnp.stack([sums[0], sums[1], sums[2], sums[3], sums[4]]) / n


# speedup vs baseline: 6.6355x; 1.0591x over previous
import jax
import jax.numpy as jnp
from jax import lax
from jax.experimental import pallas as pl
from jax.experimental.pallas import tpu as pltpu

RB = 256
CAPC = 128
NUMC_PAD = 128
JB = 4
WIDTH = 128.0
BIG = 1e30
VTH = 1e29


def _metrics_body(nrounds_ref, cls_start_ref, cls_end_ref, fc_ref, maxmb_ref,
                  dp_all_ref, dp_blk_ref, sqcol_ref,
                  out_ref,
                  a_ref, vcur_ref, voth_ref, sacc_ref,
                  minr_ref, rpn_ref, apn_ref, rm1_ref):
    i = pl.program_id(0)
    n = dp_all_ref.shape[0]
    gstart = i * RB

    g = lax.dot_general(dp_blk_ref[...], dp_all_ref[...],
                        (((1,), (1,)), ((), ())),
                        preferred_element_type=jnp.float32)
    a = sqcol_ref[...] - 2.0 * g
    col = lax.broadcasted_iota(jnp.int32, (RB, n), 1)
    row_g = gstart + lax.broadcasted_iota(jnp.int32, (RB, n), 0)
    a_ref[...] = jnp.where(col == row_g, BIG, a)

    minr_ref[...] = jnp.full((RB, 1), 1e9, jnp.float32)
    rpn_ref[...] = jnp.zeros((RB, 1), jnp.float32)
    apn_ref[...] = jnp.zeros((RB, 1), jnp.float32)

    rows_g = gstart + lax.broadcasted_iota(jnp.int32, (RB, 1), 0)
    colc2 = lax.broadcasted_iota(jnp.int32, (1, 2 * CAPC), 1)
    lanec = lax.broadcasted_iota(jnp.int32, (1, CAPC), 1)

    def extract(q, v_ref):
        v_ref[...] = jnp.full((RB, CAPC), BIG, jnp.float32)

        def cond(cls):
            return jnp.logical_and(cls < NUMC_PAD,
                                   cls_start_ref[cls] < gstart + RB)

        def body(cls):
            s = cls_start_ref[cls]
            e = cls_end_ref[cls]
            m = e - s
            base = s + q * CAPC
            cs = jnp.minimum((base // CAPC) * CAPC, n - 2 * CAPC)
            cs = pl.multiple_of(cs, CAPC)
            win = a_ref[:, pl.ds(cs, 2 * CAPC)]
            memidx = (cs - s) + colc2
            validm = jnp.logical_and(memidx >= q * CAPC,
                                     memidx < jnp.minimum(m, (q + 1) * CAPC))
            win = jnp.where(validm, win, BIG)
            offset = base - cs
            rot = pltpu.roll(win, 2 * CAPC - offset, axis=1)
            inrow = jnp.logical_and(rows_g >= s, rows_g < e)
            v_ref[...] = jnp.where(inrow, rot[:, :CAPC], v_ref[...])
            rm1_ref[...] = jnp.where(inrow, (m - 1).astype(jnp.float32),
                                     rm1_ref[...])
            return cls + 1

        lax.while_loop(cond, body, fc_ref[i])

    nrounds = nrounds_ref[0]
    maxmb = maxmb_ref[i]

    def round_body(q, _):
        numj = jnp.clip(maxmb - q * CAPC, 0, CAPC)
        extract(q, vcur_ref)

        sacc_ref[...] = jnp.zeros((RB, CAPC), jnp.float32)

        def q2_body(q2, _):
            extract(q2, voth_ref)

            def j_body(j, _):
                onehot = lanec == j
                vj = jnp.sum(jnp.where(onehot, vcur_ref[...], 0.0),
                             axis=1, keepdims=True)
                cnt = jnp.sum((voth_ref[...] < vj).astype(jnp.float32),
                              axis=1, keepdims=True)
                sacc_ref[...] += jnp.where(onehot, cnt, 0.0)
                return 0

            return lax.fori_loop(0, numj, j_body, 0)

        lax.fori_loop(0, nrounds, q2_body, 0)

        rcap = jnp.minimum(rm1_ref[...], WIDTH)

        def j_body2(jb, _):
            a = a_ref[...]
            vcur = vcur_ref[...]
            sacc = sacc_ref[...]
            for o in range(JB):
                onehot = lanec == (jb * JB + o)
                vj = jnp.sum(jnp.where(onehot, vcur, 0.0),
                             axis=1, keepdims=True)
                cnt = jnp.sum((a < vj).astype(jnp.float32),
                              axis=1, keepdims=True)
                rank = cnt + 1.0
                valid = vj < VTH
                minr_ref[...] = jnp.minimum(minr_ref[...],
                                            jnp.where(valid, rank, 1e9))
                rpn_ref[...] += jnp.where(
                    jnp.logical_and(valid, rank <= rcap + 0.5), 1.0, 0.0)
                sj = jnp.sum(jnp.where(onehot, sacc, 0.0),
                             axis=1, keepdims=True)
                apn_ref[...] += jnp.where(
                    jnp.logical_and(valid, rank <= WIDTH + 0.5),
                    (sj + 1.0) / rank, 0.0)
            return 0

        lax.fori_loop(0, (numj + JB - 1) // JB, j_body2, 0)
        return 0

    lax.fori_loop(0, nrounds, round_body, 0)

    minr = minr_ref[...]
    r1 = jnp.sum((minr <= 1.5).astype(jnp.float32))
    r5 = jnp.sum((minr <= 5.5).astype(jnp.float32))
    r10 = jnp.sum((minr <= 10.5).astype(jnp.float32))
    rp = jnp.sum(rpn_ref[...] / jnp.maximum(rm1_ref[...], 1.0))
    ap = jnp.sum(apn_ref[...]) / WIDTH

    lane = lax.broadcasted_iota(jnp.int32, (1, 128), 1)
    vec = (jnp.where(lane == 0, r1, 0.0) + jnp.where(lane == 1, r5, 0.0)
           + jnp.where(lane == 2, r10, 0.0) + jnp.where(lane == 3, rp, 0.0)
           + jnp.where(lane == 4, ap, 0.0))

    @pl.when(i == 0)
    def _():
        out_ref[...] = jnp.zeros((1, 128), jnp.float32)

    out_ref[...] += vec


@jax.jit
def kernel(d, c):
    n, dim = d.shape
    order = jnp.argsort(c)
    cp = c[order]
    dp = d[order]
    sq = jnp.sum(dp * dp, axis=1)[None, :]
    carange = jnp.arange(NUMC_PAD, dtype=cp.dtype)
    cls_start = jnp.searchsorted(cp, carange, side='left').astype(jnp.int32)
    cls_end = jnp.searchsorted(cp, carange, side='right').astype(jnp.int32)
    nb = n // RB
    fc = cp[::RB].astype(jnp.int32)
    m_row = (cls_end - cls_start)[cp]
    maxmb = jnp.max(m_row.reshape(nb, RB), axis=1).astype(jnp.int32)
    maxm = jnp.max(cls_end - cls_start)
    nrounds = jnp.maximum((maxm + CAPC - 1) // CAPC, 1).astype(jnp.int32)[None]

    out = pl.pallas_call(
        _metrics_body,
        grid=(nb,),
        in_specs=[
            pl.BlockSpec(memory_space=pltpu.SMEM),
            pl.BlockSpec(memory_space=pltpu.SMEM),
            pl.BlockSpec(memory_space=pltpu.SMEM),
            pl.BlockSpec(memory_space=pltpu.SMEM),
            pl.BlockSpec(memory_space=pltpu.SMEM),
            pl.BlockSpec((n, dim), lambda i: (0, 0)),
            pl.BlockSpec((RB, dim), lambda i: (i, 0)),
            pl.BlockSpec((1, n), lambda i: (0, 0)),
        ],
        out_specs=pl.BlockSpec((1, 128), lambda i: (0, 0)),
        out_shape=jax.ShapeDtypeStruct((1, 128), jnp.float32),
        scratch_shapes=[
            pltpu.VMEM((RB, n), jnp.float32),
            pltpu.VMEM((RB, CAPC), jnp.float32),
            pltpu.VMEM((RB, CAPC), jnp.float32),
            pltpu.VMEM((RB, CAPC), jnp.float32),
            pltpu.VMEM((RB, 1), jnp.float32),
            pltpu.VMEM((RB, 1), jnp.float32),
            pltpu.VMEM((RB, 1), jnp.float32),
            pltpu.VMEM((RB, 1), jnp.float32),
        ],
    )(nrounds, cls_start, cls_end, fc, maxmb, dp, dp, sq)

    sums = out[0]
    return jnp.stack([sums[0], sums[1], sums[2], sums[3], sums[4]]) / n


# speedup vs baseline: 31.0213x; 4.6751x over previous
import jax
import jax.numpy as jnp
from jax import lax
from jax.experimental import pallas as pl
from jax.experimental.pallas import tpu as pltpu

RB = 256
NUMC_PAD = 128
WIDTH = 128.0
BIG = 1e30
VTH = 1e29


def _metrics_body(maxst_ref, dp_all_ref, dp_blk_ref, sqcol_ref,
                  start_ref, end_ref, rm1_ref,
                  out_ref,
                  a_ref, seg_ref, prev_ref, act_ref, hc_ref,
                  minr_ref, rpn_ref, apn_ref):
    i = pl.program_id(0)
    n = dp_all_ref.shape[0]
    gstart = i * RB

    g = lax.dot_general(dp_blk_ref[...], dp_all_ref[...],
                        (((1,), (1,)), ((), ())),
                        preferred_element_type=jnp.float32)
    a = sqcol_ref[...] - 2.0 * g
    col = lax.broadcasted_iota(jnp.int32, (RB, n), 1)
    row_g = gstart + lax.broadcasted_iota(jnp.int32, (RB, n), 0)
    a = jnp.where(col == row_g, BIG, a)
    a_ref[...] = a

    inseg = jnp.logical_and(col >= start_ref[...], col < end_ref[...])
    seg_ref[...] = jnp.where(inseg, a, BIG)

    prev_ref[...] = jnp.full((RB, 1), -BIG, jnp.float32)
    act_ref[...] = jnp.ones((RB, 1), jnp.float32)
    hc_ref[...] = jnp.zeros((RB, 1), jnp.float32)
    minr_ref[...] = jnp.full((RB, 1), 1e9, jnp.float32)
    rpn_ref[...] = jnp.zeros((RB, 1), jnp.float32)
    apn_ref[...] = jnp.zeros((RB, 1), jnp.float32)

    rcap = jnp.minimum(rm1_ref[...], WIDTH)
    maxst = maxst_ref[0]

    def cond(state):
        k, anyact = state
        return jnp.logical_and(anyact, k < maxst)

    def body(state):
        k, _ = state
        cand = jnp.min(jnp.where(seg_ref[...] > prev_ref[...],
                                 seg_ref[...], BIG),
                       axis=1, keepdims=True)
        cnt = jnp.sum((a_ref[...] < cand).astype(jnp.float32),
                      axis=1, keepdims=True)
        rank = cnt + 1.0
        valid = jnp.logical_and(cand < VTH, act_ref[...] > 0.0)
        is_hit = jnp.logical_and(valid, rank <= WIDTH + 0.5)
        minr_ref[...] = jnp.minimum(minr_ref[...],
                                    jnp.where(valid, rank, 1e9))
        rpn_ref[...] += jnp.where(
            jnp.logical_and(valid, rank <= rcap + 0.5), 1.0, 0.0)
        apn_ref[...] += jnp.where(is_hit, (hc_ref[...] + 1.0) / rank, 0.0)
        hc_ref[...] += jnp.where(is_hit, 1.0, 0.0)
        prev_ref[...] = jnp.where(valid, cand, prev_ref[...])
        act = jnp.where(is_hit, 1.0, 0.0)
        act_ref[...] = act
        return k + 1, jnp.sum(act) > 0.0

    lax.while_loop(cond, body, (0, True))

    minr = minr_ref[...]
    r1 = jnp.sum((minr <= 1.5).astype(jnp.float32))
    r5 = jnp.sum((minr <= 5.5).astype(jnp.float32))
    r10 = jnp.sum((minr <= 10.5).astype(jnp.float32))
    rp = jnp.sum(rpn_ref[...] / jnp.maximum(rm1_ref[...], 1.0))
    ap = jnp.sum(apn_ref[...]) / WIDTH

    lane = lax.broadcasted_iota(jnp.int32, (1, 128), 1)
    vec = (jnp.where(lane == 0, r1, 0.0) + jnp.where(lane == 1, r5, 0.0)
           + jnp.where(lane == 2, r10, 0.0) + jnp.where(lane == 3, rp, 0.0)
           + jnp.where(lane == 4, ap, 0.0))

    @pl.when(i == 0)
    def _():
        out_ref[...] = jnp.zeros((1, 128), jnp.float32)

    out_ref[...] += vec


@jax.jit
def kernel(d, c):
    n, dim = d.shape
    order = jnp.argsort(c)
    cp = c[order]
    dp = d[order]
    sq = jnp.sum(dp * dp, axis=1)[None, :]
    carange = jnp.arange(NUMC_PAD, dtype=cp.dtype)
    cls_start = jnp.searchsorted(cp, carange, side='left').astype(jnp.int32)
    cls_end = jnp.searchsorted(cp, carange, side='right').astype(jnp.int32)
    nb = n // RB
    start_row = cls_start[cp].reshape(n, 1)
    end_row = cls_end[cp].reshape(n, 1)
    rm1_row = (end_row - start_row - 1).astype(jnp.float32)
    maxm = jnp.max(cls_end - cls_start)
    maxst = (maxm + 2).astype(jnp.int32)[None]

    out = pl.pallas_call(
        _metrics_body,
        grid=(nb,),
        in_specs=[
            pl.BlockSpec(memory_space=pltpu.SMEM),
            pl.BlockSpec((n, dim), lambda i: (0, 0)),
            pl.BlockSpec((RB, dim), lambda i: (i, 0)),
            pl.BlockSpec((1, n), lambda i: (0, 0)),
            pl.BlockSpec((RB, 1), lambda i: (i, 0)),
            pl.BlockSpec((RB, 1), lambda i: (i, 0)),
            pl.BlockSpec((RB, 1), lambda i: (i, 0)),
        ],
        out_specs=pl.BlockSpec((1, 128), lambda i: (0, 0)),
        out_shape=jax.ShapeDtypeStruct((1, 128), jnp.float32),
        scratch_shapes=[
            pltpu.VMEM((RB, n), jnp.float32),
            pltpu.VMEM((RB, n), jnp.float32),
            pltpu.VMEM((RB, 1), jnp.float32),
            pltpu.VMEM((RB, 1), jnp.float32),
            pltpu.VMEM((RB, 1), jnp.float32),
            pltpu.VMEM((RB, 1), jnp.float32),
            pltpu.VMEM((RB, 1), jnp.float32),
            pltpu.VMEM((RB, 1), jnp.float32),
        ],
    )(maxst, dp, dp, sq, start_row, end_row, rm1_row)

    sums = out[0]
    return jnp.stack([sums[0], sums[1], sums[2], sums[3], sums[4]]) / n
